# baseline (speedup 1.0000x reference)
import functools

import jax
import jax.numpy as jnp
from jax import lax
from jax.experimental import pallas as pl
from jax.experimental.pallas import tpu as pltpu
from jax.experimental.pallas import tpu_sc as plsc

_NW = 32
_LANES = 16
_TPAD = 16


def _sc_gather_mean(nodes, geo, k, feat_lin, t, d):
    b_total = nodes.shape[0]
    k_pad = geo.shape[1]
    bpw = b_total // _NW
    grp = 32
    n_grp = bpw // grp
    kt = k * t
    chunks_per_t = d // _LANES
    chunks = t * chunks_per_t
    scale = 1.0 / k

    mesh = plsc.VectorSubcoreMesh(core_axis_name="c", subcore_axis_name="s")

    @functools.partial(
        pl.kernel,
        out_type=jax.ShapeDtypeStruct((b_total, _TPAD, d), jnp.float32),
        mesh=mesh,
        scratch_types=[
            pltpu.VMEM((grp,), jnp.int32),
            pltpu.VMEM((grp,), jnp.int32),
            pltpu.VMEM((grp, k_pad), jnp.int32),
            pltpu.VMEM((grp, k_pad), jnp.int32),
            pltpu.VMEM((kt,), jnp.int32),
            pltpu.VMEM((kt,), jnp.int32),
            pltpu.VMEM((kt, d), jnp.float32),
            pltpu.VMEM((kt, d), jnp.float32),
            pltpu.VMEM((1, _TPAD, d), jnp.float32),
            pltpu.VMEM((1, _TPAD, d), jnp.float32),
            pltpu.SemaphoreType.DMA,
            pltpu.SemaphoreType.DMA,
            pltpu.SemaphoreType.DMA,
            pltpu.SemaphoreType.DMA,
            pltpu.SemaphoreType.DMA,
            pltpu.SemaphoreType.DMA,
        ],
    )
    def sc_kernel(nodes_hbm, geo_hbm, feat_hbm, out_hbm,
                  nodes_g0, nodes_g1, geo0, geo1,
                  idx0, idx1, rows0, rows1, out0, out1,
                  sem_geo0, sem_geo1, sem_g0, sem_g1, sem_o0, sem_o1):
        w = lax.axis_index("s") * 2 + lax.axis_index("c")
        base = w * bpw

        nodes_g = (nodes_g0, nodes_g1)
        geos = (geo0, geo1)
        idxs = (idx0, idx1)
        rows = (rows0, rows1)
        outs = (out0, out1)
        sems_geo = (sem_geo0, sem_geo1)
        sems_g = (sem_g0, sem_g1)
        sems_o = (sem_o0, sem_o1)

        lane = lax.iota(jnp.int32, _LANES)

        def fire_geo(g):
            gb = g % 2
            pltpu.sync_copy(nodes_hbm.at[pl.ds(base + g * grp, grp)],
                            nodes_g[gb])
            pltpu.async_copy(geo_hbm.at[nodes_g[gb]], geos[gb],
                             sems_geo[gb])

        def wait_geo(g):
            gb = g % 2
            pltpu.make_async_copy(
                geo_hbm.at[nodes_g[gb]], geos[gb], sems_geo[gb]).wait()

        def build_idx_and_fire(g, j, b):
            geo_g = geos[g % 2]
            geo_row = geo_g[j, pl.ds(0, _LANES)]
            for c in range(kt // _LANES):
                jj = lane + (c * _LANES)
                q = (jj * 2731) >> 15
                rem = jj - q * t
                gid = geo_row.at[q].get(
                    mode=lax.GatherScatterMode.PROMISE_IN_BOUNDS)
                idxs[b][pl.ds(c * _LANES, _LANES)] = gid * t + rem
            pltpu.async_copy(feat_hbm.at[idxs[b]], rows[b], sems_g[b])

        def wait_rows(b):
            pltpu.make_async_copy(
                feat_hbm.at[idxs[b]], rows[b], sems_g[b]).wait()

        def wait_out(i_global, b):
            pltpu.make_async_copy(
                outs[b], out_hbm.at[pl.ds(base + i_global, 1)],
                sems_o[b]).wait()

        def reduce_node(b):
            def chunk(j, carry):
                tt = j // chunks_per_t
                o = (j % chunks_per_t) * _LANES
                acc = rows[b][tt, pl.ds(o, _LANES)]
                for r in range(1, k):
                    acc = acc + rows[b][r * t + tt, pl.ds(o, _LANES)]
                outs[b][0, tt, pl.ds(o, _LANES)] = acc * scale
                return carry
            lax.fori_loop(0, chunks, chunk, 0, unroll=8)

        fire_geo(0)
        fire_geo(1)

        for g in range(n_grp):
            wait_geo(g)
            build_idx_and_fire(g, 0, 0)
            build_idx_and_fire(g, 1, 1)

            def pair(p, carry):
                for b in range(2):
                    j = 2 * p + b
                    i_global = g * grp + j
                    wait_rows(b)
                    if g == 0:
                        @pl.when(i_global >= 2)
                        def _():
                            wait_out(i_global - 2, b)
                    else:
                        wait_out(i_global - 2, b)
                    reduce_node(b)
                    pltpu.async_copy(
                        outs[b], out_hbm.at[pl.ds(base + i_global, 1)],
                        sems_o[b])

                    @pl.when(j + 2 < grp)
                    def _():
                        build_idx_and_fire(g, j + 2, b)
                return carry

            lax.fori_loop(0, grp // 2, pair, 0)

            if g + 2 < n_grp:
                fire_geo(g + 2)

        wait_out(bpw - 2, 0)
        wait_out(bpw - 1, 1)

    return sc_kernel(nodes, geo, feat_lin)


def _tc_matmul_relu(features, neigh_pad, weight, block_b=128):
    b, t, d = features.shape
    e = weight.shape[0]

    def body(x1_ref, x2_ref, w_ref, o_ref):
        w1 = w_ref[:, :d].astype(jnp.bfloat16)
        w2 = w_ref[:, d:].astype(jnp.bfloat16)
        dn = (((1,), (1,)), ((), ()))
        for tt in range(t):
            x1 = x1_ref[:, tt, :].astype(jnp.bfloat16)
            x2 = x2_ref[:, tt, :].astype(jnp.bfloat16)
            acc = lax.dot_general(x1, w1, dn,
                                  preferred_element_type=jnp.float32)
            acc = acc + lax.dot_general(x2, w2, dn,
                                        preferred_element_type=jnp.float32)
            o_ref[:, tt, :] = jnp.maximum(acc, 0.0)

    return pl.pallas_call(
        body,
        grid=(b // block_b,),
        in_specs=[
            pl.BlockSpec((block_b, t, d), lambda i: (i, 0, 0)),
            pl.BlockSpec((block_b, _TPAD, d), lambda i: (i, 0, 0)),
            pl.BlockSpec((e, 2 * d), lambda i: (0, 0)),
        ],
        out_specs=pl.BlockSpec((block_b, t, e), lambda i: (i, 0, 0)),
        out_shape=jax.ShapeDtypeStruct((b, t, e), jnp.float32),
    )(features, neigh_pad, weight)


def kernel(features, feat_out, nodes, geo_neighbors, weight):
    b, t, d = features.shape
    n = feat_out.shape[0]

    nodes_i = nodes.astype(jnp.int32)
    k = geo_neighbors.shape[1]
    geo_pad = jnp.pad(geo_neighbors, ((0, 0), (0, 128 - k)))
    feat_lin = feat_out.reshape(n * t, d)
    neigh_pad = _sc_gather_mean(nodes_i, geo_pad, k, feat_lin, t, d)

    return _tc_matmul_relu(features, neigh_pad, weight)


# speedup vs baseline: 1.4224x; 1.4224x over previous
import functools

import jax
import jax.numpy as jnp
from jax import lax
from jax.experimental import pallas as pl
from jax.experimental.pallas import tpu as pltpu
from jax.experimental.pallas import tpu_sc as plsc

_NW = 32
_LANES = 16


def _sc_gather_mean(nodes, geo, k, feat2d):
    b_total = nodes.shape[0]
    k_pad = geo.shape[1]
    td = feat2d.shape[1]
    bpw = b_total // _NW
    chunks = td // _LANES
    scale = 1.0 / k

    mesh = plsc.VectorSubcoreMesh(core_axis_name="c", subcore_axis_name="s")

    @functools.partial(
        pl.kernel,
        out_type=jax.ShapeDtypeStruct((b_total, td), jnp.float32),
        mesh=mesh,
        scratch_types=[
            pltpu.VMEM((bpw,), jnp.int32),
            pltpu.VMEM((bpw, k_pad), jnp.int32),
            pltpu.VMEM((k, td), jnp.float32),
            pltpu.VMEM((k, td), jnp.float32),
            pltpu.VMEM((1, td), jnp.float32),
            pltpu.VMEM((1, td), jnp.float32),
            pltpu.SemaphoreType.DMA,
            pltpu.SemaphoreType.DMA,
            pltpu.SemaphoreType.DMA,
            pltpu.SemaphoreType.DMA,
        ],
    )
    def sc_kernel(nodes_hbm, geo_hbm, feat_hbm, out_hbm,
                  nodes_v, geo_v, rows0, rows1, out0, out1,
                  sem_g0, sem_g1, sem_o0, sem_o1):
        w = lax.axis_index("s") * 2 + lax.axis_index("c")
        base = w * bpw

        pltpu.sync_copy(nodes_hbm.at[pl.ds(base, bpw)], nodes_v)
        pltpu.async_copy(geo_hbm.at[nodes_v], geo_v, sem_g0).wait()

        rows = (rows0, rows1)
        outs = (out0, out1)
        sems_g = (sem_g0, sem_g1)
        sems_o = (sem_o0, sem_o1)

        def fire_gather(i, b):
            pltpu.async_copy(feat_hbm.at[geo_v.at[i, pl.ds(0, k)]],
                             rows[b], sems_g[b])

        fire_gather(0, 0)
        fire_gather(1, 1)

        def process(i, b):
            pltpu.make_async_copy(
                feat_hbm.at[geo_v.at[i, pl.ds(0, k)]], rows[b],
                sems_g[b]).wait()

            @pl.when(i >= 2)
            def _():
                pltpu.make_async_copy(
                    outs[b], out_hbm.at[pl.ds(base + i - 2, 1)],
                    sems_o[b]).wait()

            def chunk(j, carry):
                o = j * _LANES
                acc = rows[b][0, pl.ds(o, _LANES)]
                for r in range(1, k):
                    acc = acc + rows[b][r, pl.ds(o, _LANES)]
                outs[b][0, pl.ds(o, _LANES)] = acc * scale
                return carry

            lax.fori_loop(0, chunks, chunk, 0, unroll=8)

            pltpu.async_copy(outs[b], out_hbm.at[pl.ds(base + i, 1)],
                             sems_o[b])

            @pl.when(i + 2 < bpw)
            def _():
                fire_gather(i + 2, b)

        def outer(g, carry):
            process(2 * g, 0)
            process(2 * g + 1, 1)
            return carry

        lax.fori_loop(0, bpw // 2, outer, 0)

        pltpu.make_async_copy(
            out0, out_hbm.at[pl.ds(base + bpw - 2, 1)], sem_o0).wait()
        pltpu.make_async_copy(
            out1, out_hbm.at[pl.ds(base + bpw - 1, 1)], sem_o1).wait()

    return sc_kernel(nodes, geo, feat2d)


def _tc_matmul_relu(features, neigh, weight, block_b=128):
    b, t, d = features.shape
    e = weight.shape[0]

    def body(x1_ref, x2_ref, w_ref, o_ref):
        w1 = w_ref[:, :d]
        w2 = w_ref[:, d:]
        dn = (((1,), (1,)), ((), ()))
        for tt in range(t):
            x1 = x1_ref[:, tt, :]
            x2 = x2_ref[:, pl.ds(tt * d, d)]
            acc = lax.dot_general(x1, w1, dn,
                                  preferred_element_type=jnp.float32)
            acc = acc + lax.dot_general(x2, w2, dn,
                                        preferred_element_type=jnp.float32)
            o_ref[:, tt, :] = jnp.maximum(acc, 0.0)

    return pl.pallas_call(
        body,
        grid=(b // block_b,),
        in_specs=[
            pl.BlockSpec((block_b, t, d), lambda i: (i, 0, 0)),
            pl.BlockSpec((block_b, t * d), lambda i: (i, 0)),
            pl.BlockSpec((e, 2 * d), lambda i: (0, 0)),
        ],
        out_specs=pl.BlockSpec((block_b, t, e), lambda i: (i, 0, 0)),
        out_shape=jax.ShapeDtypeStruct((b, t, e), jnp.float32),
    )(features, neigh, weight)


def kernel(features, feat_out, nodes, geo_neighbors, weight):
    b, t, d = features.shape
    n = feat_out.shape[0]

    nodes_i = nodes.astype(jnp.int32)
    k = geo_neighbors.shape[1]
    geo_pad = jnp.pad(geo_neighbors, ((0, 0), (0, 128 - k)))
    feat2d = feat_out.reshape(n, t * d)
    neigh = _sc_gather_mean(nodes_i, geo_pad, k, feat2d)

    return _tc_matmul_relu(features, neigh, weight)


# speedup vs baseline: 1.6230x; 1.1410x over previous
import functools

import jax
import jax.numpy as jnp
from jax import lax
from jax.experimental import pallas as pl
from jax.experimental.pallas import tpu as pltpu
from jax.experimental.pallas import tpu_sc as plsc

_NW = 32
_LANES = 16


def _sc_gather_mean(nodes, geo, k, feat2d):
    b_total = nodes.shape[0]
    k_pad = geo.shape[1]
    td = feat2d.shape[1]
    bpw = b_total // _NW
    chunks = td // _LANES
    scale = 1.0 / k

    mesh = plsc.VectorSubcoreMesh(core_axis_name="c", subcore_axis_name="s")

    @functools.partial(
        pl.kernel,
        out_type=jax.ShapeDtypeStruct((b_total, td), jnp.float32),
        mesh=mesh,
        scratch_types=[
            pltpu.VMEM((bpw,), jnp.int32),
            pltpu.VMEM((bpw, k_pad), jnp.int32),
            pltpu.VMEM((k, td), jnp.float32),
            pltpu.VMEM((k, td), jnp.float32),
            pltpu.VMEM((1, td), jnp.float32),
            pltpu.VMEM((1, td), jnp.float32),
            pltpu.SemaphoreType.DMA,
            pltpu.SemaphoreType.DMA,
            pltpu.SemaphoreType.DMA,
            pltpu.SemaphoreType.DMA,
        ],
    )
    def sc_kernel(nodes_hbm, geo_hbm, feat_hbm, out_hbm,
                  nodes_v, geo_v, rows0, rows1, out0, out1,
                  sem_g0, sem_g1, sem_o0, sem_o1):
        w = lax.axis_index("s") * 2 + lax.axis_index("c")
        base = w * bpw

        pltpu.sync_copy(nodes_hbm.at[pl.ds(base, bpw)], nodes_v)
        pltpu.async_copy(geo_hbm.at[nodes_v], geo_v, sem_g0).wait()

        rows = (rows0, rows1)
        outs = (out0, out1)
        sems_g = (sem_g0, sem_g1)
        sems_o = (sem_o0, sem_o1)

        def fire_gather(i, b):
            pltpu.async_copy(feat_hbm.at[geo_v.at[i, pl.ds(0, k)]],
                             rows[b], sems_g[b])

        fire_gather(0, 0)
        fire_gather(1, 1)

        def process(i, b):
            pltpu.make_async_copy(
                feat_hbm.at[geo_v.at[i, pl.ds(0, k)]], rows[b],
                sems_g[b]).wait()

            @pl.when(i >= 2)
            def _():
                pltpu.make_async_copy(
                    outs[b], out_hbm.at[pl.ds(base + i - 2, 1)],
                    sems_o[b]).wait()

            def chunk(j, carry):
                o = j * _LANES
                acc = rows[b][0, pl.ds(o, _LANES)]
                for r in range(1, k):
                    acc = acc + rows[b][r, pl.ds(o, _LANES)]
                outs[b][0, pl.ds(o, _LANES)] = acc * scale
                return carry

            lax.fori_loop(0, chunks, chunk, 0, unroll=8)

            pltpu.async_copy(outs[b], out_hbm.at[pl.ds(base + i, 1)],
                             sems_o[b])

            @pl.when(i + 2 < bpw)
            def _():
                fire_gather(i + 2, b)

        def outer(g, carry):
            process(2 * g, 0)
            process(2 * g + 1, 1)
            return carry

        lax.fori_loop(0, bpw // 2, outer, 0)

        pltpu.make_async_copy(
            out0, out_hbm.at[pl.ds(base + bpw - 2, 1)], sem_o0).wait()
        pltpu.make_async_copy(
            out1, out_hbm.at[pl.ds(base + bpw - 1, 1)], sem_o1).wait()

    return sc_kernel(nodes, geo, feat2d)


def _tc_matmul_relu_chunk(features, neigh_c, weight, chunk, n_chunks,
                          out_prev, block_b=128):
    b, t, d = features.shape
    e = weight.shape[0]
    blocks = b // n_chunks // block_b
    base = chunk * blocks

    def body(x1_ref, x2_ref, w_ref, *rest):
        o_ref = rest[-1]
        w1 = w_ref[:, :d]
        w2 = w_ref[:, d:]
        dn = (((1,), (1,)), ((), ()))
        for tt in range(t):
            x1 = x1_ref[:, tt, :]
            x2 = x2_ref[:, pl.ds(tt * d, d)]
            acc = lax.dot_general(x1, w1, dn,
                                  preferred_element_type=jnp.float32)
            acc = acc + lax.dot_general(x2, w2, dn,
                                        preferred_element_type=jnp.float32)
            o_ref[:, tt, :] = jnp.maximum(acc, 0.0)

    in_specs = [
        pl.BlockSpec((block_b, t, d), lambda i: (base + i, 0, 0)),
        pl.BlockSpec((block_b, t * d), lambda i: (i, 0)),
        pl.BlockSpec((e, 2 * d), lambda i: (0, 0)),
    ]
    operands = [features, neigh_c, weight]
    aliases = {}
    if out_prev is not None:
        in_specs.append(pl.BlockSpec(memory_space=pl.ANY))
        operands.append(out_prev)
        aliases = {3: 0}

    return pl.pallas_call(
        body,
        grid=(blocks,),
        in_specs=in_specs,
        out_specs=pl.BlockSpec((block_b, t, e), lambda i: (base + i, 0, 0)),
        out_shape=jax.ShapeDtypeStruct((b, t, e), jnp.float32),
        input_output_aliases=aliases,
    )(*operands)


def kernel(features, feat_out, nodes, geo_neighbors, weight):
    b, t, d = features.shape
    n = feat_out.shape[0]

    nodes_i = nodes.astype(jnp.int32)
    k = geo_neighbors.shape[1]
    geo_pad = jnp.pad(geo_neighbors, ((0, 0), (0, 128 - k)))
    feat2d = feat_out.reshape(n, t * d)

    n_chunks = 4
    bc = b // n_chunks
    neighs = [_sc_gather_mean(lax.dynamic_slice_in_dim(nodes_i, c * bc, bc),
                              geo_pad, k, feat2d)
              for c in range(n_chunks)]

    out = None
    for c in range(n_chunks):
        out = _tc_matmul_relu_chunk(features, neighs[c], weight,
                                    c, n_chunks, out)
    return out


# speedup vs baseline: 2.8678x; 1.7670x over previous
import functools

import jax
import jax.numpy as jnp
from jax import lax
from jax.experimental import pallas as pl
from jax.experimental.pallas import tpu as pltpu
from jax.experimental.pallas import tpu_sc as plsc

_NW = 32
_LANES = 16


def _sc_gather_mean(nodes, geo, k, feat_tmaj, n_rows, t, d):
    b_total = nodes.shape[0]
    k_pad = geo.shape[1]
    td = t * d
    bpw = b_total // _NW
    kt = k * t
    chunks_per_t = d // _LANES
    chunks = t * chunks_per_t
    scale = 1.0 / k

    mesh = plsc.VectorSubcoreMesh(core_axis_name="c", subcore_axis_name="s")

    @functools.partial(
        pl.kernel,
        out_type=jax.ShapeDtypeStruct((b_total, td), jnp.float32),
        mesh=mesh,
        scratch_types=[
            pltpu.VMEM((bpw,), jnp.int32),
            pltpu.VMEM((bpw, k_pad), jnp.int32),
            pltpu.VMEM((kt,), jnp.int32),
            pltpu.VMEM((kt,), jnp.int32),
            pltpu.VMEM((kt, d), jnp.float32),
            pltpu.VMEM((kt, d), jnp.float32),
            pltpu.VMEM((1, td), jnp.float32),
            pltpu.VMEM((1, td), jnp.float32),
            pltpu.SemaphoreType.DMA,
            pltpu.SemaphoreType.DMA,
            pltpu.SemaphoreType.DMA,
            pltpu.SemaphoreType.DMA,
        ],
    )
    def sc_kernel(nodes_hbm, geo_hbm, feat_hbm, out_hbm,
                  nodes_v, geo_v, idx0, idx1, rows0, rows1, out0, out1,
                  sem_g0, sem_g1, sem_o0, sem_o1):
        w = lax.axis_index("s") * 2 + lax.axis_index("c")
        base = w * bpw

        pltpu.sync_copy(nodes_hbm.at[pl.ds(base, bpw)], nodes_v)
        pltpu.async_copy(geo_hbm.at[nodes_v], geo_v, sem_g0).wait()

        idxs = (idx0, idx1)
        rows = (rows0, rows1)
        outs = (out0, out1)
        sems_g = (sem_g0, sem_g1)
        sems_o = (sem_o0, sem_o1)

        lane = lax.iota(jnp.int32, _LANES)

        def build_idx_and_fire(i, b):
            geo_row = geo_v[i, pl.ds(0, _LANES)]
            for c in range(kt // _LANES):
                jj = lane + (c * _LANES)
                m = jj & (k - 1)
                p = jj >> 3
                gid = geo_row.at[m].get(
                    mode=lax.GatherScatterMode.PROMISE_IN_BOUNDS)
                idxs[b][pl.ds(c * _LANES, _LANES)] = gid + p * n_rows
            pltpu.async_copy(feat_hbm.at[idxs[b]], rows[b], sems_g[b])

        build_idx_and_fire(0, 0)
        build_idx_and_fire(1, 1)

        def process(i, b):
            pltpu.make_async_copy(
                feat_hbm.at[idxs[b]], rows[b], sems_g[b]).wait()

            @pl.when(i >= 2)
            def _():
                pltpu.make_async_copy(
                    outs[b], out_hbm.at[pl.ds(base + i - 2, 1)],
                    sems_o[b]).wait()

            def chunk(j, carry):
                p = j // chunks_per_t
                o = (j % chunks_per_t) * _LANES
                r0 = p * k
                acc = rows[b][r0, pl.ds(o, _LANES)]
                for m in range(1, k):
                    acc = acc + rows[b][r0 + m, pl.ds(o, _LANES)]
                outs[b][0, pl.ds(p * d + o, _LANES)] = acc * scale
                return carry

            lax.fori_loop(0, chunks, chunk, 0, unroll=8)

            pltpu.async_copy(outs[b], out_hbm.at[pl.ds(base + i, 1)],
                             sems_o[b])

            @pl.when(i + 2 < bpw)
            def _():
                build_idx_and_fire(i + 2, b)

        def pair(g, carry):
            process(2 * g, 0)
            process(2 * g + 1, 1)
            return carry

        lax.fori_loop(0, bpw // 2, pair, 0)

        pltpu.make_async_copy(
            out0, out_hbm.at[pl.ds(base + bpw - 2, 1)], sem_o0).wait()
        pltpu.make_async_copy(
            out1, out_hbm.at[pl.ds(base + bpw - 1, 1)], sem_o1).wait()

    return sc_kernel(nodes, geo, feat_tmaj)


def _tc_matmul_relu_chunk(feat_tmaj3, neigh_c, weight, chunk, n_chunks,
                          out_prev, block_b=512):
    t, b, d = feat_tmaj3.shape
    e = weight.shape[0]
    blocks = b // n_chunks // block_b
    base = chunk * blocks

    def body(x1_ref, x2_ref, w_ref, *rest):
        o_ref = rest[-1]
        w1 = w_ref[:, :d]
        w2 = w_ref[:, d:]
        dn = (((1,), (1,)), ((), ()))
        acc = lax.dot_general(x1_ref[0], w1, dn,
                              preferred_element_type=jnp.float32)
        acc = acc + lax.dot_general(x2_ref[...], w2, dn,
                                    preferred_element_type=jnp.float32)
        o_ref[0] = jnp.maximum(acc, 0.0)

    in_specs = [
        pl.BlockSpec((1, block_b, d), lambda i, tt: (tt, base + i, 0)),
        pl.BlockSpec((block_b, d), lambda i, tt: (i, tt)),
        pl.BlockSpec((e, 2 * d), lambda i, tt: (0, 0)),
    ]
    operands = [feat_tmaj3, neigh_c, weight]
    aliases = {}
    if out_prev is not None:
        in_specs.append(pl.BlockSpec(memory_space=pl.ANY))
        operands.append(out_prev)
        aliases = {3: 0}

    return pl.pallas_call(
        body,
        grid=(blocks, t),
        in_specs=in_specs,
        out_specs=pl.BlockSpec((1, block_b, e),
                               lambda i, tt: (tt, base + i, 0)),
        out_shape=jax.ShapeDtypeStruct((t, b, e), jnp.float32),
        input_output_aliases=aliases,
    )(*operands)


def kernel(features, feat_out, nodes, geo_neighbors, weight):
    b, t, d = features.shape
    n = feat_out.shape[0]

    nodes_i = nodes.astype(jnp.int32)
    k = geo_neighbors.shape[1]
    geo_pad = jnp.pad(geo_neighbors, ((0, 0), (0, 128 - k)))
    feat_tmaj = jnp.transpose(feat_out, (1, 0, 2)).reshape(t * n, d)
    feat_tmaj3 = jnp.transpose(features, (1, 0, 2))

    n_chunks = 4
    bc = b // n_chunks
    neighs = [_sc_gather_mean(lax.dynamic_slice_in_dim(nodes_i, c * bc, bc),
                              geo_pad, k, feat_tmaj, n, t, d)
              for c in range(n_chunks)]

    out = None
    for c in range(n_chunks):
        out = _tc_matmul_relu_chunk(feat_tmaj3, neighs[c], weight,
                                    c, n_chunks, out)
    return jnp.transpose(out, (1, 0, 2))
